# baseline (speedup 1.0000x reference)
import jax
import jax.numpy as jnp
from jax.experimental import pallas as pl

_BN = 1024


def _gconv_lstm_body(x_ref, w_ref, bias_ref, wco_ref, wlin_ref, blin_ref,
                     out_ref, h_ref, c_ref):
    d = h_ref.shape[1]
    g = jnp.dot(x_ref[...], w_ref[...],
                preferred_element_type=jnp.float32) + bias_ref[...]
    i_gate = jax.nn.sigmoid(g[:, 0:d])
    t_gate = jnp.tanh(g[:, d:2 * d])
    c = i_gate * t_gate
    o_gate = jax.nn.sigmoid(g[:, 2 * d:3 * d] + wco_ref[...] * c)
    h = o_gate * jnp.tanh(c)
    c_ref[...] = c
    h_ref[...] = h
    out_ref[...] = jnp.dot(jnp.maximum(h, 0.0), wlin_ref[...],
                           preferred_element_type=jnp.float32) + blin_ref[...]


def kernel(x, edge_index, edge_attr, W_x_i, b_x_i, W_h_i, b_h_i, b_i, w_c_i,
           W_x_f, b_x_f, W_h_f, b_h_f, b_f, w_c_f, W_x_c, b_x_c, W_h_c,
           b_h_c, b_c, W_x_o, b_x_o, W_h_o, b_h_o, b_o, w_c_o, W_lin, b_lin):
    del edge_index, edge_attr
    del W_h_i, W_h_f, W_h_c, W_h_o, w_c_i
    del W_x_f, b_x_f, b_h_f, b_f, w_c_f

    n, f_in = x.shape
    d = W_x_i.shape[1]

    w_cat = jnp.concatenate([W_x_i, W_x_c, W_x_o], axis=1)
    bias_cat = jnp.concatenate([
        (b_x_i + b_h_i)[None, :] + b_i,
        (b_x_c + b_h_c)[None, :] + b_c,
        (b_x_o + b_h_o)[None, :] + b_o,
    ], axis=1)
    blin = b_lin.reshape(1, 1)

    grid = (pl.cdiv(n, _BN),)
    row_spec = lambda shp: pl.BlockSpec(shp, lambda idx: (idx, 0))
    full_spec = lambda shp: pl.BlockSpec(shp, lambda idx: (0, 0))

    out, h, c = pl.pallas_call(
        _gconv_lstm_body,
        grid=grid,
        in_specs=[
            row_spec((_BN, f_in)),
            full_spec((f_in, 3 * d)),
            full_spec((1, 3 * d)),
            full_spec((1, d)),
            full_spec((d, 1)),
            full_spec((1, 1)),
        ],
        out_specs=[
            row_spec((_BN, 1)),
            row_spec((_BN, d)),
            row_spec((_BN, d)),
        ],
        out_shape=[
            jax.ShapeDtypeStruct((n, 1), x.dtype),
            jax.ShapeDtypeStruct((n, d), x.dtype),
            jax.ShapeDtypeStruct((n, d), x.dtype),
        ],
    )(x, w_cat, bias_cat, w_c_o, W_lin, blin)
    return (out, h, c)


# speedup vs baseline: 1.0022x; 1.0022x over previous
import jax
import jax.numpy as jnp
from jax.experimental import pallas as pl
from jax.experimental.pallas import tpu as pltpu

_BN = 1024


def _gconv_lstm_body(x_ref, w_ref, bias_ref, wco_ref, wlin_ref, blin_ref,
                     out_ref, h_ref, c_ref):
    d = h_ref.shape[1]
    g = jnp.dot(x_ref[...], w_ref[...],
                preferred_element_type=jnp.float32) + bias_ref[...]
    i_gate = jax.nn.sigmoid(g[:, 0:d])
    t_gate = jnp.tanh(g[:, d:2 * d])
    c = i_gate * t_gate
    o_gate = jax.nn.sigmoid(g[:, 2 * d:3 * d] + wco_ref[...] * c)
    h = o_gate * jnp.tanh(c)
    c_ref[...] = c
    h_ref[...] = h
    out_ref[...] = jnp.dot(jnp.maximum(h, 0.0), wlin_ref[...],
                           preferred_element_type=jnp.float32) + blin_ref[...]


def kernel(x, edge_index, edge_attr, W_x_i, b_x_i, W_h_i, b_h_i, b_i, w_c_i,
           W_x_f, b_x_f, W_h_f, b_h_f, b_f, w_c_f, W_x_c, b_x_c, W_h_c,
           b_h_c, b_c, W_x_o, b_x_o, W_h_o, b_h_o, b_o, w_c_o, W_lin, b_lin):
    del edge_index, edge_attr
    del W_h_i, W_h_f, W_h_c, W_h_o, w_c_i
    del W_x_f, b_x_f, b_h_f, b_f, w_c_f

    n, f_in = x.shape
    d = W_x_i.shape[1]

    w_cat = jnp.concatenate([W_x_i, W_x_c, W_x_o], axis=1)
    bias_cat = jnp.concatenate([
        (b_x_i + b_h_i)[None, :] + b_i,
        (b_x_c + b_h_c)[None, :] + b_c,
        (b_x_o + b_h_o)[None, :] + b_o,
    ], axis=1)
    blin = b_lin.reshape(1, 1)

    grid = (pl.cdiv(n, _BN),)
    row_spec = lambda shp: pl.BlockSpec(shp, lambda idx: (idx, 0))
    full_spec = lambda shp: pl.BlockSpec(shp, lambda idx: (0, 0))

    out, h, c = pl.pallas_call(
        _gconv_lstm_body,
        grid=grid,
        in_specs=[
            row_spec((_BN, f_in)),
            full_spec((f_in, 3 * d)),
            full_spec((1, 3 * d)),
            full_spec((1, d)),
            full_spec((d, 1)),
            full_spec((1, 1)),
        ],
        out_specs=[
            row_spec((_BN, 1)),
            row_spec((_BN, d)),
            row_spec((_BN, d)),
        ],
        out_shape=[
            jax.ShapeDtypeStruct((n, 1), x.dtype),
            jax.ShapeDtypeStruct((n, d), x.dtype),
            jax.ShapeDtypeStruct((n, d), x.dtype),
        ],
        compiler_params=pltpu.CompilerParams(
            dimension_semantics=("parallel",)),
    )(x, w_cat, bias_cat, w_c_o, W_lin, blin)
    return (out, h, c)


# speedup vs baseline: 1.0695x; 1.0672x over previous
import jax
import jax.numpy as jnp
from jax.experimental import pallas as pl
from jax.experimental.pallas import tpu as pltpu

_BN = 2048


def _gconv_lstm_body(x_ref, w_ref, bias_ref, wco_ref, wlin_ref, blin_ref,
                     out_ref, h_ref, c_ref):
    d = h_ref.shape[1]
    g = jnp.dot(x_ref[...], w_ref[...],
                preferred_element_type=jnp.float32) + bias_ref[...]
    i_gate = jax.nn.sigmoid(g[:, 0:d])
    t_gate = jnp.tanh(g[:, d:2 * d])
    c = i_gate * t_gate
    o_gate = jax.nn.sigmoid(g[:, 2 * d:3 * d] + wco_ref[...] * c)
    h = o_gate * jnp.tanh(c)
    c_ref[...] = c
    h_ref[...] = h
    out_ref[...] = jnp.dot(jnp.maximum(h, 0.0), wlin_ref[...],
                           preferred_element_type=jnp.float32) + blin_ref[...]


def kernel(x, edge_index, edge_attr, W_x_i, b_x_i, W_h_i, b_h_i, b_i, w_c_i,
           W_x_f, b_x_f, W_h_f, b_h_f, b_f, w_c_f, W_x_c, b_x_c, W_h_c,
           b_h_c, b_c, W_x_o, b_x_o, W_h_o, b_h_o, b_o, w_c_o, W_lin, b_lin):
    del edge_index, edge_attr
    del W_h_i, W_h_f, W_h_c, W_h_o, w_c_i
    del W_x_f, b_x_f, b_h_f, b_f, w_c_f

    n, f_in = x.shape
    d = W_x_i.shape[1]

    w_cat = jnp.concatenate([W_x_i, W_x_c, W_x_o], axis=1)
    bias_cat = jnp.concatenate([
        (b_x_i + b_h_i)[None, :] + b_i,
        (b_x_c + b_h_c)[None, :] + b_c,
        (b_x_o + b_h_o)[None, :] + b_o,
    ], axis=1)
    blin = b_lin.reshape(1, 1)

    grid = (pl.cdiv(n, _BN),)
    row_spec = lambda shp: pl.BlockSpec(shp, lambda idx: (idx, 0))
    full_spec = lambda shp: pl.BlockSpec(shp, lambda idx: (0, 0))

    out, h, c = pl.pallas_call(
        _gconv_lstm_body,
        grid=grid,
        in_specs=[
            row_spec((_BN, f_in)),
            full_spec((f_in, 3 * d)),
            full_spec((1, 3 * d)),
            full_spec((1, d)),
            full_spec((d, 1)),
            full_spec((1, 1)),
        ],
        out_specs=[
            row_spec((_BN, 1)),
            row_spec((_BN, d)),
            row_spec((_BN, d)),
        ],
        out_shape=[
            jax.ShapeDtypeStruct((n, 1), x.dtype),
            jax.ShapeDtypeStruct((n, d), x.dtype),
            jax.ShapeDtypeStruct((n, d), x.dtype),
        ],
        compiler_params=pltpu.CompilerParams(
            dimension_semantics=("parallel",)),
    )(x, w_cat, bias_cat, w_c_o, W_lin, blin)
    return (out, h, c)
